# Initial kernel scaffold; baseline (speedup 1.0000x reference)
import jax
import jax.numpy as jnp
from jax.experimental import pallas as pl


def kernel(inds, num_bins):
    raise NotImplementedError("write your pallas kernel here")


# speedup vs baseline: 1.5939x; 1.5939x over previous
import jax
import jax.numpy as jnp
from jax import lax
from jax.experimental import pallas as pl
from jax.experimental.pallas import tpu as pltpu
from jax.experimental.pallas import tpu_sc as plsc

_N = 8388608
_NB = 100000
_NBP = 100096
_NC = 2
_NS = 16
_NW = _NC * _NS
_PER_TILE = _N // _NW
_CH = 4096
_NCHUNK = _PER_TILE // _CH
_SLICE = _NBP // _NS
_G = 2


def _sc_hist_body(inds_hbm, out_hbm, idx_a, idx_b, hist, tmp, shared,
                  sem_a, sem_b):
    cid = lax.axis_index("c")
    sid = lax.axis_index("s")
    wid = cid * _NS + sid
    base = wid * _PER_TILE

    zeros = jnp.zeros((16,), jnp.int32)
    ones = jnp.full((16,), 1, jnp.int32)

    def zero_body(i, carry):
        hist[pl.ds(i * 16, 16)] = zeros
        return carry
    lax.fori_loop(0, _NBP // 16, zero_body, 0, unroll=8)

    def scatter_chunk(idx_ref):
        def body(r, carry):
            iv = idx_ref[pl.ds(r * 16, 16)]
            plsc.addupdate_scatter(hist, [iv], ones)
            return carry
        lax.fori_loop(0, _CH // 16, body, 0, unroll=8)

    bufs = (idx_a, idx_b)
    sems = (sem_a, sem_b)
    pltpu.async_copy(inds_hbm.at[pl.ds(base, _CH)], idx_a, sem_a).wait()
    for c in range(_NCHUNK):
        cur = bufs[c % 2]
        if c + 1 < _NCHUNK:
            nxt = bufs[(c + 1) % 2]
            cp = pltpu.async_copy(
                inds_hbm.at[pl.ds(base + (c + 1) * _CH, _CH)], nxt,
                sems[(c + 1) % 2])
        scatter_chunk(cur)
        if c + 1 < _NCHUNK:
            cp.wait()

    off = sid * _SLICE
    for r in range(_NS // _G):
        plsc.subcore_barrier()

        @pl.when(sid // _G == r)
        def _publish():
            pltpu.sync_copy(hist, shared.at[pl.ds((sid % _G) * _NBP, _NBP)])

        plsc.subcore_barrier()
        for t in range(_G):
            gt = r * _G + t

            @pl.when(sid != gt)
            def _accum():
                pltpu.sync_copy(shared.at[pl.ds(t * _NBP + off, _SLICE)],
                                tmp)

                def add_body(v, carry):
                    plsc.addupdate(hist.at[pl.ds(off + v * 16, 16)],
                                   tmp[pl.ds(v * 16, 16)])
                    return carry
                lax.fori_loop(0, _SLICE // 16, add_body, 0, unroll=8)

    pltpu.sync_copy(hist.at[pl.ds(off, _SLICE)],
                    out_hbm.at[pl.ds(cid * _NBP + off, _SLICE)])


_sc_hist = pl.kernel(
    _sc_hist_body,
    out_type=jax.ShapeDtypeStruct((_NC * _NBP,), jnp.int32),
    mesh=plsc.VectorSubcoreMesh(core_axis_name="c", subcore_axis_name="s"),
    scratch_types=[
        pltpu.VMEM((_CH,), jnp.int32),
        pltpu.VMEM((_CH,), jnp.int32),
        pltpu.VMEM((_NBP,), jnp.int32),
        pltpu.VMEM((_SLICE,), jnp.int32),
        pltpu.VMEM_SHARED((_G * _NBP,), jnp.int32),
        pltpu.SemaphoreType.DMA,
        pltpu.SemaphoreType.DMA,
    ],
    compiler_params=pltpu.CompilerParams(needs_layout_passes=False),
)


def _tc_merge_body(ha_ref, hb_ref, cnt_ref, s_ref, ss_ref):
    h = ha_ref[...] + hb_ref[...]
    cnt_ref[...] = h
    r = lax.broadcasted_iota(jnp.int32, (_NBP // 128, 128), 0)
    c = lax.broadcasted_iota(jnp.int32, (_NBP // 128, 128), 1)
    b = r * 128 + c
    s_ref[0, 0] = jnp.sum(h * b)
    ss_ref[0, 0] = jnp.sum(h * (b * b))


_tc_merge = pl.pallas_call(
    _tc_merge_body,
    out_shape=[
        jax.ShapeDtypeStruct((_NBP // 128, 128), jnp.int32),
        jax.ShapeDtypeStruct((1, 1), jnp.int32),
        jax.ShapeDtypeStruct((1, 1), jnp.int32),
    ],
    out_specs=[
        pl.BlockSpec(memory_space=pltpu.VMEM),
        pl.BlockSpec(memory_space=pltpu.SMEM),
        pl.BlockSpec(memory_space=pltpu.SMEM),
    ],
)


def kernel(inds, num_bins):
    parts = _sc_hist(inds)
    ha = parts[:_NBP].reshape(_NBP // 128, 128)
    hb = parts[_NBP:].reshape(_NBP // 128, 128)
    cnt2d, s, ss = _tc_merge(ha, hb)
    counts = cnt2d.reshape(-1)[:_NB]
    limits = jnp.arange(_NB + 1, dtype=jnp.int32)
    hist_min = jnp.asarray(0, jnp.int32)
    hist_max = jnp.asarray(num_bins - 1, jnp.int32)
    num = jnp.asarray(_N, jnp.int32)
    return (hist_min, hist_max, num, s[0, 0], ss[0, 0], limits, counts)


# speedup vs baseline: 2.2460x; 1.4091x over previous
import jax
import jax.numpy as jnp
from jax import lax
from jax.experimental import pallas as pl
from jax.experimental.pallas import tpu as pltpu
from jax.experimental.pallas import tpu_sc as plsc

_N = 8388608
_NB = 100000
_NBP = 100352
_NC = 2
_NS = 16
_NW = _NC * _NS
_PER_TILE = _N // _NW
_CH = 4096
_NCHUNK = _PER_TILE // _CH
_SLICE = _NBP // _NS
_G = 2


def _sc_hist_body(inds_hbm, out_hbm, idx_a, idx_b, hist, tmp, shared,
                  sem_a, sem_b):
    cid = lax.axis_index("c")
    sid = lax.axis_index("s")
    wid = cid * _NS + sid
    base = wid * _PER_TILE

    zeros = jnp.zeros((16,), jnp.int32)
    ones = jnp.full((16,), 1, jnp.int32)

    def zero_body(i, carry):
        hist[pl.ds(i * 16, 16)] = zeros
        return carry
    lax.fori_loop(0, _NBP // 16, zero_body, 0, unroll=8)

    def scatter_chunk(idx_ref):
        def body(r, carry):
            ivs = [idx_ref[pl.ds((r * 8 + k) * 16, 16)] for k in range(8)]
            for iv in ivs:
                plsc.addupdate_scatter(hist, [iv], ones)
            return carry
        lax.fori_loop(0, _CH // 128, body, 0, unroll=2)

    bufs = (idx_a, idx_b)
    sems = (sem_a, sem_b)
    pltpu.async_copy(inds_hbm.at[pl.ds(base, _CH)], idx_a, sem_a).wait()
    for c in range(_NCHUNK):
        cur = bufs[c % 2]
        if c + 1 < _NCHUNK:
            nxt = bufs[(c + 1) % 2]
            cp = pltpu.async_copy(
                inds_hbm.at[pl.ds(base + (c + 1) * _CH, _CH)], nxt,
                sems[(c + 1) % 2])
        scatter_chunk(cur)
        if c + 1 < _NCHUNK:
            cp.wait()

    off = sid * _SLICE
    for r in range(_NS // _G):
        plsc.subcore_barrier()

        @pl.when(sid // _G == r)
        def _publish():
            pltpu.sync_copy(hist, shared.at[pl.ds((sid % _G) * _NBP, _NBP)])

        plsc.subcore_barrier()
        for t in range(_G):
            gt = r * _G + t

            @pl.when(sid != gt)
            def _accum():
                pltpu.sync_copy(shared.at[pl.ds(t * _NBP + off, _SLICE)],
                                tmp)

                def add_body(v, carry):
                    vals = [tmp[pl.ds((v * 8 + k) * 16, 16)]
                            for k in range(8)]
                    for k in range(8):
                        plsc.addupdate(
                            hist.at[pl.ds(off + (v * 8 + k) * 16, 16)],
                            vals[k])
                    return carry
                lax.fori_loop(0, _SLICE // 128, add_body, 0, unroll=2)

    pltpu.sync_copy(hist.at[pl.ds(off, _SLICE)],
                    out_hbm.at[pl.ds(cid * _NBP + off, _SLICE)])


_sc_hist = pl.kernel(
    _sc_hist_body,
    out_type=jax.ShapeDtypeStruct((_NC * _NBP,), jnp.int32),
    mesh=plsc.VectorSubcoreMesh(core_axis_name="c", subcore_axis_name="s"),
    scratch_types=[
        pltpu.VMEM((_CH,), jnp.int32),
        pltpu.VMEM((_CH,), jnp.int32),
        pltpu.VMEM((_NBP,), jnp.int32),
        pltpu.VMEM((_SLICE,), jnp.int32),
        pltpu.VMEM_SHARED((_G * _NBP,), jnp.int32),
        pltpu.SemaphoreType.DMA,
        pltpu.SemaphoreType.DMA,
    ],
    compiler_params=pltpu.CompilerParams(needs_layout_passes=False),
)


def _tc_merge_body(ha_ref, hb_ref, cnt_ref, s_ref, ss_ref):
    h = ha_ref[...] + hb_ref[...]
    cnt_ref[...] = h
    r = lax.broadcasted_iota(jnp.int32, (_NBP // 128, 128), 0)
    c = lax.broadcasted_iota(jnp.int32, (_NBP // 128, 128), 1)
    b = r * 128 + c
    s_ref[0, 0] = jnp.sum(h * b)
    ss_ref[0, 0] = jnp.sum(h * (b * b))


_tc_merge = pl.pallas_call(
    _tc_merge_body,
    out_shape=[
        jax.ShapeDtypeStruct((_NBP // 128, 128), jnp.int32),
        jax.ShapeDtypeStruct((1, 1), jnp.int32),
        jax.ShapeDtypeStruct((1, 1), jnp.int32),
    ],
    out_specs=[
        pl.BlockSpec(memory_space=pltpu.VMEM),
        pl.BlockSpec(memory_space=pltpu.SMEM),
        pl.BlockSpec(memory_space=pltpu.SMEM),
    ],
)


def kernel(inds, num_bins):
    parts = _sc_hist(inds)
    ha = parts[:_NBP].reshape(_NBP // 128, 128)
    hb = parts[_NBP:].reshape(_NBP // 128, 128)
    cnt2d, s, ss = _tc_merge(ha, hb)
    counts = cnt2d.reshape(-1)[:_NB]
    limits = jnp.arange(_NB + 1, dtype=jnp.int32)
    hist_min = jnp.asarray(0, jnp.int32)
    hist_max = jnp.asarray(num_bins - 1, jnp.int32)
    num = jnp.asarray(_N, jnp.int32)
    return (hist_min, hist_max, num, s[0, 0], ss[0, 0], limits, counts)


# speedup vs baseline: 3.1026x; 1.3814x over previous
import jax
import jax.numpy as jnp
from jax import lax
from jax.experimental import pallas as pl
from jax.experimental.pallas import tpu as pltpu
from jax.experimental.pallas import tpu_sc as plsc

_N = 8388608
_NB = 100000
_NBP = 100352
_NC = 2
_NS = 16
_NW = _NC * _NS
_PER_TILE = _N // _NW
_CH = 4096
_NCHUNK = _PER_TILE // _CH


def _sc_hist_body(inds_hbm, out_hbm, idx_a, idx_b, hist, sem_a, sem_b):
    cid = lax.axis_index("c")
    sid = lax.axis_index("s")
    wid = cid * _NS + sid
    base = wid * _PER_TILE

    zeros = jnp.zeros((16,), jnp.int32)
    ones = jnp.full((16,), 1, jnp.int32)

    def zero_body(i, carry):
        hist[pl.ds(i * 16, 16)] = zeros
        return carry
    lax.fori_loop(0, _NBP // 16, zero_body, 0, unroll=8)

    def scatter_chunk(idx_ref):
        def body(r, carry):
            ivs = [idx_ref[pl.ds((r * 8 + k) * 16, 16)] for k in range(8)]
            for iv in ivs:
                plsc.addupdate_scatter(hist, [iv], ones)
            return carry
        lax.fori_loop(0, _CH // 128, body, 0, unroll=2)

    bufs = (idx_a, idx_b)
    sems = (sem_a, sem_b)
    pltpu.async_copy(inds_hbm.at[pl.ds(base, _CH)], idx_a, sem_a).wait()
    for c in range(_NCHUNK):
        cur = bufs[c % 2]
        if c + 1 < _NCHUNK:
            nxt = bufs[(c + 1) % 2]
            cp = pltpu.async_copy(
                inds_hbm.at[pl.ds(base + (c + 1) * _CH, _CH)], nxt,
                sems[(c + 1) % 2])
        scatter_chunk(cur)
        if c + 1 < _NCHUNK:
            cp.wait()

    pltpu.sync_copy(hist, out_hbm.at[pl.ds(wid * _NBP, _NBP)])


_sc_hist = pl.kernel(
    _sc_hist_body,
    out_type=jax.ShapeDtypeStruct((_NW * _NBP,), jnp.int32),
    mesh=plsc.VectorSubcoreMesh(core_axis_name="c", subcore_axis_name="s"),
    scratch_types=[
        pltpu.VMEM((_CH,), jnp.int32),
        pltpu.VMEM((_CH,), jnp.int32),
        pltpu.VMEM((_NBP,), jnp.int32),
        pltpu.SemaphoreType.DMA,
        pltpu.SemaphoreType.DMA,
    ],
    compiler_params=pltpu.CompilerParams(needs_layout_passes=False),
)


def _tc_merge_body(h_ref, cnt_ref, lim_ref, s_ref, ss_ref):
    h = jnp.sum(h_ref[...], axis=0)
    cnt_ref[...] = h
    r = lax.broadcasted_iota(jnp.int32, (_NBP // 128, 128), 0)
    c = lax.broadcasted_iota(jnp.int32, (_NBP // 128, 128), 1)
    b = r * 128 + c
    lim_ref[...] = b
    s_ref[0, 0] = jnp.sum(h * b)
    ss_ref[0, 0] = jnp.sum(h * (b * b))


_tc_merge = pl.pallas_call(
    _tc_merge_body,
    out_shape=[
        jax.ShapeDtypeStruct((_NBP // 128, 128), jnp.int32),
        jax.ShapeDtypeStruct((_NBP // 128, 128), jnp.int32),
        jax.ShapeDtypeStruct((1, 1), jnp.int32),
        jax.ShapeDtypeStruct((1, 1), jnp.int32),
    ],
    out_specs=[
        pl.BlockSpec(memory_space=pltpu.VMEM),
        pl.BlockSpec(memory_space=pltpu.VMEM),
        pl.BlockSpec(memory_space=pltpu.SMEM),
        pl.BlockSpec(memory_space=pltpu.SMEM),
    ],
)


def kernel(inds, num_bins):
    parts = _sc_hist(inds)
    h3 = parts.reshape(_NW, _NBP // 128, 128)
    cnt2d, lim2d, s, ss = _tc_merge(h3)
    counts = cnt2d.reshape(-1)[:_NB]
    limits = lim2d.reshape(-1)[:_NB + 1]
    hist_min = jnp.asarray(0, jnp.int32)
    hist_max = jnp.asarray(num_bins - 1, jnp.int32)
    num = jnp.asarray(_N, jnp.int32)
    return (hist_min, hist_max, num, s[0, 0], ss[0, 0], limits, counts)


# speedup vs baseline: 4.3985x; 1.4177x over previous
import jax
import jax.numpy as jnp
from jax import lax
from jax.experimental import pallas as pl
from jax.experimental.pallas import tpu as pltpu
from jax.experimental.pallas import tpu_sc as plsc

_N = 8388608
_NB = 100000
_NBP = 100352
_NC = 2
_NS = 16
_NW = _NC * _NS
_PER_TILE = _N // _NW
_CH = 4096
_NCHUNK = _PER_TILE // _CH


def _sc_hist_body(inds_hbm, out_hbm, idx_a, idx_b, idx_c, idx_d, hist,
                  sem_a, sem_b, sem_c, sem_d):
    cid = lax.axis_index("c")
    sid = lax.axis_index("s")
    wid = cid * _NS + sid
    base = wid * _PER_TILE

    zeros = jnp.zeros((16,), jnp.int32)
    ones = jnp.full((16,), 1, jnp.int32)

    def zero_body(i, carry):
        hist[pl.ds(i * 16, 16)] = zeros
        return carry
    lax.fori_loop(0, _NBP // 16, zero_body, 0, unroll=8)

    def scatter_chunk(idx_ref):
        def body(r, carry):
            ivs = [idx_ref[pl.ds((r * 8 + k) * 16, 16)] for k in range(8)]
            for iv in ivs:
                plsc.addupdate_scatter(hist, [iv], ones)
            return carry
        lax.fori_loop(0, _CH // 128, body, 0, unroll=2)

    bufs = (idx_a, idx_b, idx_c, idx_d)
    sems = (sem_a, sem_b, sem_c, sem_d)
    nbuf = len(bufs)

    def issue(c):
        return pltpu.async_copy(
            inds_hbm.at[pl.ds(base + c * _CH, _CH)], bufs[c % nbuf],
            sems[c % nbuf])

    descs = {}
    for c in range(min(nbuf - 1, _NCHUNK)):
        descs[c] = issue(c)
    for c in range(_NCHUNK):
        if c + nbuf - 1 < _NCHUNK:
            descs[c + nbuf - 1] = issue(c + nbuf - 1)
        descs.pop(c).wait()
        scatter_chunk(bufs[c % nbuf])

    pltpu.sync_copy(hist, out_hbm.at[pl.ds(wid * _NBP, _NBP)])


_sc_hist = pl.kernel(
    _sc_hist_body,
    out_type=jax.ShapeDtypeStruct((_NW * _NBP,), jnp.int32),
    mesh=plsc.VectorSubcoreMesh(core_axis_name="c", subcore_axis_name="s"),
    scratch_types=[
        pltpu.VMEM((_CH,), jnp.int32),
        pltpu.VMEM((_CH,), jnp.int32),
        pltpu.VMEM((_CH,), jnp.int32),
        pltpu.VMEM((_CH,), jnp.int32),
        pltpu.VMEM((_NBP,), jnp.int32),
        pltpu.SemaphoreType.DMA,
        pltpu.SemaphoreType.DMA,
        pltpu.SemaphoreType.DMA,
        pltpu.SemaphoreType.DMA,
    ],
    compiler_params=pltpu.CompilerParams(needs_layout_passes=False),
)


def _tc_merge_body(h_ref, cnt_ref, lim_ref, s_ref, ss_ref):
    h = jnp.sum(h_ref[...], axis=0)
    cnt_ref[...] = h
    r = lax.broadcasted_iota(jnp.int32, (_NBP // 128, 128), 0)
    c = lax.broadcasted_iota(jnp.int32, (_NBP // 128, 128), 1)
    b = r * 128 + c
    lim_ref[...] = b
    s_ref[0, 0] = jnp.sum(h * b)
    ss_ref[0, 0] = jnp.sum(h * (b * b))


_tc_merge = pl.pallas_call(
    _tc_merge_body,
    out_shape=[
        jax.ShapeDtypeStruct((_NBP // 128, 128), jnp.int32),
        jax.ShapeDtypeStruct((_NBP // 128, 128), jnp.int32),
        jax.ShapeDtypeStruct((1, 1), jnp.int32),
        jax.ShapeDtypeStruct((1, 1), jnp.int32),
    ],
    out_specs=[
        pl.BlockSpec(memory_space=pltpu.VMEM),
        pl.BlockSpec(memory_space=pltpu.VMEM),
        pl.BlockSpec(memory_space=pltpu.SMEM),
        pl.BlockSpec(memory_space=pltpu.SMEM),
    ],
)


def kernel(inds, num_bins):
    parts = _sc_hist(inds)
    h3 = parts.reshape(_NW, _NBP // 128, 128)
    cnt2d, lim2d, s, ss = _tc_merge(h3)
    counts = cnt2d.reshape(-1)[:_NB]
    limits = lim2d.reshape(-1)[:_NB + 1]
    hist_min = jnp.asarray(0, jnp.int32)
    hist_max = jnp.asarray(num_bins - 1, jnp.int32)
    num = jnp.asarray(_N, jnp.int32)
    return (hist_min, hist_max, num, s[0, 0], ss[0, 0], limits, counts)


# speedup vs baseline: 4.4075x; 1.0020x over previous
import jax
import jax.numpy as jnp
from jax import lax
from jax.experimental import pallas as pl
from jax.experimental.pallas import tpu as pltpu
from jax.experimental.pallas import tpu_sc as plsc

_N = 8388608
_NB = 100000
_NBP = 100352
_NC = 2
_NS = 16
_NW = _NC * _NS
_PER_TILE = _N // _NW
_CH = 4096
_NCHUNK = _PER_TILE // _CH


def _sc_hist_body(inds_hbm, out_hbm, idx_a, idx_b, idx_c, idx_d, idx_e,
                  idx_f, hist, sem_a, sem_b, sem_c, sem_d, sem_e, sem_f):
    cid = lax.axis_index("c")
    sid = lax.axis_index("s")
    wid = cid * _NS + sid
    base = wid * _PER_TILE

    zeros = jnp.zeros((16,), jnp.int32)
    ones = jnp.full((16,), 1, jnp.int32)

    def zero_body(i, carry):
        hist[pl.ds(i * 16, 16)] = zeros
        return carry
    lax.fori_loop(0, _NBP // 16, zero_body, 0, unroll=8)

    def scatter_chunk(idx_ref):
        def body(r, carry):
            ivs = [idx_ref[pl.ds((r * 8 + k) * 16, 16)] for k in range(8)]
            for iv in ivs:
                plsc.addupdate_scatter(hist, [iv], ones)
            return carry
        lax.fori_loop(0, _CH // 128, body, 0, unroll=2)

    bufs = (idx_a, idx_b, idx_c, idx_d, idx_e, idx_f)
    sems = (sem_a, sem_b, sem_c, sem_d, sem_e, sem_f)
    nbuf = len(bufs)

    def issue(c):
        return pltpu.async_copy(
            inds_hbm.at[pl.ds(base + c * _CH, _CH)], bufs[c % nbuf],
            sems[c % nbuf])

    descs = {}
    for c in range(min(nbuf - 1, _NCHUNK)):
        descs[c] = issue(c)
    for c in range(_NCHUNK):
        if c + nbuf - 1 < _NCHUNK:
            descs[c + nbuf - 1] = issue(c + nbuf - 1)
        descs.pop(c).wait()
        scatter_chunk(bufs[c % nbuf])

    pltpu.sync_copy(hist, out_hbm.at[pl.ds(wid * _NBP, _NBP)])


_sc_hist = pl.kernel(
    _sc_hist_body,
    out_type=jax.ShapeDtypeStruct((_NW * _NBP,), jnp.int32),
    mesh=plsc.VectorSubcoreMesh(core_axis_name="c", subcore_axis_name="s"),
    scratch_types=[
        pltpu.VMEM((_CH,), jnp.int32),
        pltpu.VMEM((_CH,), jnp.int32),
        pltpu.VMEM((_CH,), jnp.int32),
        pltpu.VMEM((_CH,), jnp.int32),
        pltpu.VMEM((_CH,), jnp.int32),
        pltpu.VMEM((_CH,), jnp.int32),
        pltpu.VMEM((_NBP,), jnp.int32),
        pltpu.SemaphoreType.DMA,
        pltpu.SemaphoreType.DMA,
        pltpu.SemaphoreType.DMA,
        pltpu.SemaphoreType.DMA,
        pltpu.SemaphoreType.DMA,
        pltpu.SemaphoreType.DMA,
    ],
    compiler_params=pltpu.CompilerParams(needs_layout_passes=False),
)


def _tc_merge_body(h_ref, cnt_ref, lim_ref, s_ref, ss_ref):
    h = jnp.sum(h_ref[...], axis=0)
    cnt_ref[...] = h
    r = lax.broadcasted_iota(jnp.int32, (_NBP // 128, 128), 0)
    c = lax.broadcasted_iota(jnp.int32, (_NBP // 128, 128), 1)
    b = r * 128 + c
    lim_ref[...] = b
    s_ref[0, 0] = jnp.sum(h * b)
    ss_ref[0, 0] = jnp.sum(h * (b * b))


_tc_merge = pl.pallas_call(
    _tc_merge_body,
    out_shape=[
        jax.ShapeDtypeStruct((_NBP // 128, 128), jnp.int32),
        jax.ShapeDtypeStruct((_NBP // 128, 128), jnp.int32),
        jax.ShapeDtypeStruct((1, 1), jnp.int32),
        jax.ShapeDtypeStruct((1, 1), jnp.int32),
    ],
    out_specs=[
        pl.BlockSpec(memory_space=pltpu.VMEM),
        pl.BlockSpec(memory_space=pltpu.VMEM),
        pl.BlockSpec(memory_space=pltpu.SMEM),
        pl.BlockSpec(memory_space=pltpu.SMEM),
    ],
)


def kernel(inds, num_bins):
    parts = _sc_hist(inds)
    h3 = parts.reshape(_NW, _NBP // 128, 128)
    cnt2d, lim2d, s, ss = _tc_merge(h3)
    counts = cnt2d.reshape(-1)[:_NB]
    limits = lim2d.reshape(-1)[:_NB + 1]
    hist_min = jnp.asarray(0, jnp.int32)
    hist_max = jnp.asarray(num_bins - 1, jnp.int32)
    num = jnp.asarray(_N, jnp.int32)
    return (hist_min, hist_max, num, s[0, 0], ss[0, 0], limits, counts)


# speedup vs baseline: 5.0498x; 1.1457x over previous
import jax
import jax.numpy as jnp
from jax import lax
from jax.experimental import pallas as pl
from jax.experimental.pallas import tpu as pltpu
from jax.experimental.pallas import tpu_sc as plsc

_N = 8388608
_NB = 100000
_NBP = 100352
_NC = 2
_NS = 16
_NW = _NC * _NS
_PER_TILE = _N // _NW
_CH = 4096
_NCHUNK = _PER_TILE // _CH


def _sc_hist_body(inds_hbm, out_hbm, idx_a, idx_b, idx_c, idx_d, hist,
                  sem_a, sem_b, sem_c, sem_d):
    cid = lax.axis_index("c")
    sid = lax.axis_index("s")
    wid = cid * _NS + sid
    base = wid * _PER_TILE

    zeros = jnp.zeros((16,), jnp.int32)
    ones = jnp.full((16,), 1, jnp.int32)

    def zero_body(i, carry):
        hist[pl.ds(i * 16, 16)] = zeros
        return carry
    lax.fori_loop(0, _NBP // 16, zero_body, 0, unroll=8)

    def scatter_chunk(idx_ref):
        def body(r, carry):
            ivs = [idx_ref[pl.ds((r * 8 + k) * 16, 16)] for k in range(8)]
            for iv in ivs:
                plsc.addupdate_scatter(hist, [iv], ones)
            return carry
        lax.fori_loop(0, _CH // 128, body, 0, unroll=2)

    bufs = (idx_a, idx_b, idx_c, idx_d)
    sems = (sem_a, sem_b, sem_c, sem_d)
    nbuf = len(bufs)

    def issue(c):
        return pltpu.async_copy(
            inds_hbm.at[pl.ds(base + c * _CH, _CH)], bufs[c % nbuf],
            sems[c % nbuf])

    for c in range(nbuf - 1):
        issue(c)

    def group_body(g, carry):
        for j in range(nbuf):
            c = g * nbuf + j

            jp = (j + nbuf - 1) % nbuf

            @pl.when(c + nbuf - 1 < _NCHUNK)
            def _prefetch():
                pltpu.async_copy(
                    inds_hbm.at[pl.ds(base + (c + nbuf - 1) * _CH, _CH)],
                    bufs[jp], sems[jp])

            pltpu.make_async_copy(
                inds_hbm.at[pl.ds(base, _CH)], bufs[j], sems[j]).wait()
            scatter_chunk(bufs[j])
        return carry
    lax.fori_loop(0, _NCHUNK // nbuf, group_body, 0, unroll=1)

    pltpu.sync_copy(hist, out_hbm.at[pl.ds(wid * _NBP, _NBP)])


_sc_hist = pl.kernel(
    _sc_hist_body,
    out_type=jax.ShapeDtypeStruct((_NW * _NBP,), jnp.int32),
    mesh=plsc.VectorSubcoreMesh(core_axis_name="c", subcore_axis_name="s"),
    scratch_types=[
        pltpu.VMEM((_CH,), jnp.int32),
        pltpu.VMEM((_CH,), jnp.int32),
        pltpu.VMEM((_CH,), jnp.int32),
        pltpu.VMEM((_CH,), jnp.int32),
        pltpu.VMEM((_NBP,), jnp.int32),
        pltpu.SemaphoreType.DMA,
        pltpu.SemaphoreType.DMA,
        pltpu.SemaphoreType.DMA,
        pltpu.SemaphoreType.DMA,
    ],
    compiler_params=pltpu.CompilerParams(needs_layout_passes=False),
)


def _tc_merge_body(h_ref, cnt_ref, lim_ref, s_ref, ss_ref):
    h = jnp.sum(h_ref[...], axis=0)
    cnt_ref[...] = h
    r = lax.broadcasted_iota(jnp.int32, (_NBP // 128, 128), 0)
    c = lax.broadcasted_iota(jnp.int32, (_NBP // 128, 128), 1)
    b = r * 128 + c
    lim_ref[...] = b
    s_ref[0, 0] = jnp.sum(h * b)
    ss_ref[0, 0] = jnp.sum(h * (b * b))


_tc_merge = pl.pallas_call(
    _tc_merge_body,
    out_shape=[
        jax.ShapeDtypeStruct((_NBP // 128, 128), jnp.int32),
        jax.ShapeDtypeStruct((_NBP // 128, 128), jnp.int32),
        jax.ShapeDtypeStruct((1, 1), jnp.int32),
        jax.ShapeDtypeStruct((1, 1), jnp.int32),
    ],
    out_specs=[
        pl.BlockSpec(memory_space=pltpu.VMEM),
        pl.BlockSpec(memory_space=pltpu.VMEM),
        pl.BlockSpec(memory_space=pltpu.SMEM),
        pl.BlockSpec(memory_space=pltpu.SMEM),
    ],
)


def kernel(inds, num_bins):
    parts = _sc_hist(inds)
    h3 = parts.reshape(_NW, _NBP // 128, 128)
    cnt2d, lim2d, s, ss = _tc_merge(h3)
    counts = cnt2d.reshape(-1)[:_NB]
    limits = lim2d.reshape(-1)[:_NB + 1]
    hist_min = jnp.asarray(0, jnp.int32)
    hist_max = jnp.asarray(num_bins - 1, jnp.int32)
    num = jnp.asarray(_N, jnp.int32)
    return (hist_min, hist_max, num, s[0, 0], ss[0, 0], limits, counts)


# speedup vs baseline: 5.1024x; 1.0104x over previous
import jax
import jax.numpy as jnp
from jax import lax
from jax.experimental import pallas as pl
from jax.experimental.pallas import tpu as pltpu
from jax.experimental.pallas import tpu_sc as plsc

_N = 8388608
_NB = 100000
_NBP = 100352
_NC = 2
_NS = 16
_NW = _NC * _NS
_PER_TILE = _N // _NW
_CH = 4096
_NCHUNK = _PER_TILE // _CH


def _sc_hist_body(inds_hbm, out_hbm, idx_a, idx_b, idx_c, idx_d, idx_e,
                  hist, sem_a, sem_b, sem_c, sem_d, sem_e):
    cid = lax.axis_index("c")
    sid = lax.axis_index("s")
    wid = cid * _NS + sid
    base = wid * _PER_TILE

    zeros = jnp.zeros((16,), jnp.int32)
    ones = jnp.full((16,), 1, jnp.int32)

    def scatter_chunk(idx_ref):
        def body(r, carry):
            ivs = [idx_ref[pl.ds((r * 8 + k) * 16, 16)] for k in range(8)]
            for iv in ivs:
                plsc.addupdate_scatter(hist, [iv], ones)
            return carry
        lax.fori_loop(0, _CH // 128, body, 0, unroll=2)

    bufs = (idx_a, idx_b, idx_c, idx_d, idx_e)
    sems = (sem_a, sem_b, sem_c, sem_d, sem_e)
    nbuf = len(bufs)

    def issue(c):
        return pltpu.async_copy(
            inds_hbm.at[pl.ds(base + c * _CH, _CH)], bufs[c % nbuf],
            sems[c % nbuf])

    for c in range(nbuf - 1):
        issue(c)

    def zero_body(i, carry):
        hist[pl.ds(i * 16, 16)] = zeros
        return carry
    lax.fori_loop(0, _NBP // 16, zero_body, 0, unroll=8)

    ngroups = _NCHUNK // nbuf
    ntail = _NCHUNK - ngroups * nbuf

    def group_body(g, carry):
        for j in range(nbuf):
            c = g * nbuf + j
            jp = (j + nbuf - 1) % nbuf

            @pl.when(c + nbuf - 1 < _NCHUNK)
            def _prefetch():
                pltpu.async_copy(
                    inds_hbm.at[pl.ds(base + (c + nbuf - 1) * _CH, _CH)],
                    bufs[jp], sems[jp])

            pltpu.make_async_copy(
                inds_hbm.at[pl.ds(base, _CH)], bufs[j], sems[j]).wait()
            scatter_chunk(bufs[j])
        return carry
    lax.fori_loop(0, ngroups, group_body, 0, unroll=1)

    for c in range(ngroups * nbuf, _NCHUNK):
        j = c % nbuf
        pltpu.make_async_copy(
            inds_hbm.at[pl.ds(base, _CH)], bufs[j], sems[j]).wait()
        scatter_chunk(bufs[j])

    pltpu.sync_copy(hist, out_hbm.at[pl.ds(wid * _NBP, _NBP)])


_sc_hist = pl.kernel(
    _sc_hist_body,
    out_type=jax.ShapeDtypeStruct((_NW * _NBP,), jnp.int32),
    mesh=plsc.VectorSubcoreMesh(core_axis_name="c", subcore_axis_name="s"),
    scratch_types=[
        pltpu.VMEM((_CH,), jnp.int32),
        pltpu.VMEM((_CH,), jnp.int32),
        pltpu.VMEM((_CH,), jnp.int32),
        pltpu.VMEM((_CH,), jnp.int32),
        pltpu.VMEM((_CH,), jnp.int32),
        pltpu.VMEM((_NBP,), jnp.int32),
        pltpu.SemaphoreType.DMA,
        pltpu.SemaphoreType.DMA,
        pltpu.SemaphoreType.DMA,
        pltpu.SemaphoreType.DMA,
        pltpu.SemaphoreType.DMA,
    ],
    compiler_params=pltpu.CompilerParams(needs_layout_passes=False),
)


def _tc_merge_body(h_ref, cnt_ref, lim_ref, s_ref, ss_ref):
    h = jnp.sum(h_ref[...], axis=0)
    cnt_ref[...] = h
    r = lax.broadcasted_iota(jnp.int32, (_NBP // 128, 128), 0)
    c = lax.broadcasted_iota(jnp.int32, (_NBP // 128, 128), 1)
    b = r * 128 + c
    lim_ref[...] = b
    s_ref[0, 0] = jnp.sum(h * b)
    ss_ref[0, 0] = jnp.sum(h * (b * b))


_tc_merge = pl.pallas_call(
    _tc_merge_body,
    out_shape=[
        jax.ShapeDtypeStruct((_NBP // 128, 128), jnp.int32),
        jax.ShapeDtypeStruct((_NBP // 128, 128), jnp.int32),
        jax.ShapeDtypeStruct((1, 1), jnp.int32),
        jax.ShapeDtypeStruct((1, 1), jnp.int32),
    ],
    out_specs=[
        pl.BlockSpec(memory_space=pltpu.VMEM),
        pl.BlockSpec(memory_space=pltpu.VMEM),
        pl.BlockSpec(memory_space=pltpu.SMEM),
        pl.BlockSpec(memory_space=pltpu.SMEM),
    ],
)


def kernel(inds, num_bins):
    parts = _sc_hist(inds)
    h3 = parts.reshape(_NW, _NBP // 128, 128)
    cnt2d, lim2d, s, ss = _tc_merge(h3)
    counts = cnt2d.reshape(-1)[:_NB]
    limits = lim2d.reshape(-1)[:_NB + 1]
    hist_min = jnp.asarray(0, jnp.int32)
    hist_max = jnp.asarray(num_bins - 1, jnp.int32)
    num = jnp.asarray(_N, jnp.int32)
    return (hist_min, hist_max, num, s[0, 0], ss[0, 0], limits, counts)
